# baseline (speedup 1.0000x reference)
import functools

import jax
import jax.numpy as jnp
from jax import lax
from jax.experimental import pallas as pl
from jax.experimental.pallas import tpu as pltpu
from jax.experimental.pallas import tpu_sc as plsc

VOCAB = 1000
BATCH = 16384

_NUM_CORES = 2
_NUM_SUBCORES = 16
_NW = _NUM_CORES * _NUM_SUBCORES
_B_PER_W = BATCH // _NW
_CHUNK = 64
_N_CHUNKS = _B_PER_W // _CHUNK


def _make_emb_kernel():
    mesh = plsc.VectorSubcoreMesh(core_axis_name="c", subcore_axis_name="s")

    @functools.partial(
        pl.kernel,
        mesh=mesh,
        out_type=jax.ShapeDtypeStruct((BATCH, VOCAB), jnp.float32),
        compiler_params=pltpu.CompilerParams(use_tc_tiling_on_sc=False),
        scratch_types=[
            pltpu.VMEM((_B_PER_W,), jnp.int32),
            pltpu.VMEM((_CHUNK, VOCAB), jnp.float32),
            pltpu.VMEM((_CHUNK, VOCAB), jnp.float32),
            pltpu.SemaphoreType.DMA,
            pltpu.SemaphoreType.DMA,
            pltpu.SemaphoreType.DMA,
            pltpu.SemaphoreType.DMA,
        ],
    )
    def emb_kernel(idx_hbm, table_hbm, out_hbm,
                   idx_v, rows0, rows1, g0, g1, s0, s1):
        wid = lax.axis_index("s") * _NUM_CORES + lax.axis_index("c")
        base = wid * _B_PER_W
        pltpu.sync_copy(idx_hbm.at[pl.ds(base, _B_PER_W)], idx_v)

        bufs = (rows0, rows1)
        gsems = (g0, g1)
        ssems = (s0, s1)

        def gather(c):
            p = c % 2
            h = pltpu.make_async_copy(
                table_hbm.at[idx_v.at[pl.ds(c * _CHUNK, _CHUNK)]],
                bufs[p], gsems[p])
            h.start()
            return h

        def store(c):
            p = c % 2
            h = pltpu.make_async_copy(
                bufs[p], out_hbm.at[pl.ds(base + c * _CHUNK, _CHUNK)],
                ssems[p])
            h.start()
            return h

        g_h = [None] * _N_CHUNKS
        s_h = [None] * _N_CHUNKS
        g_h[0] = gather(0)
        g_h[1] = gather(1)
        for c in range(_N_CHUNKS):
            g_h[c].wait()
            s_h[c] = store(c)
            if c + 2 < _N_CHUNKS:
                s_h[c].wait()
                g_h[c + 2] = gather(c + 2)
        s_h[_N_CHUNKS - 2].wait()
        s_h[_N_CHUNKS - 1].wait()

    return emb_kernel


_emb_lookup = _make_emb_kernel()


def kernel(idx, embedding_table):
    idx1 = idx.reshape(-1).astype(jnp.int32)
    return _emb_lookup(idx1, embedding_table)


# speedup vs baseline: 1.5775x; 1.5775x over previous
import functools

import jax
import jax.numpy as jnp
from jax import lax
from jax.experimental import pallas as pl
from jax.experimental.pallas import tpu as pltpu
from jax.experimental.pallas import tpu_sc as plsc

VOCAB = 1000
BATCH = 16384
DPAD = 1024

_NUM_CORES = 2
_NUM_SUBCORES = 16
_NW = _NUM_CORES * _NUM_SUBCORES
_B_PER_W = BATCH // _NW
_CHUNK = 32
_N_CHUNKS = _B_PER_W // _CHUNK


def _make_emb_kernel():
    mesh = plsc.VectorSubcoreMesh(core_axis_name="c", subcore_axis_name="s")

    @functools.partial(
        pl.kernel,
        mesh=mesh,
        out_type=jax.ShapeDtypeStruct((BATCH, DPAD), jnp.float32),
        compiler_params=pltpu.CompilerParams(use_tc_tiling_on_sc=True),
        scratch_types=[
            pltpu.VMEM((_B_PER_W,), jnp.int32),
            pltpu.VMEM((_CHUNK, DPAD), jnp.float32),
            pltpu.VMEM((_CHUNK, DPAD), jnp.float32),
            pltpu.SemaphoreType.DMA,
            pltpu.SemaphoreType.DMA,
            pltpu.SemaphoreType.DMA,
            pltpu.SemaphoreType.DMA,
        ],
    )
    def emb_kernel(idx_hbm, table_hbm, out_hbm,
                   idx_v, rows0, rows1, g0, g1, s0, s1):
        wid = lax.axis_index("s") * _NUM_CORES + lax.axis_index("c")
        base = wid * _B_PER_W
        pltpu.sync_copy(idx_hbm.at[pl.ds(base, _B_PER_W)], idx_v)

        bufs = (rows0, rows1)
        gsems = (g0, g1)
        ssems = (s0, s1)

        def gather(c):
            p = c % 2
            h = pltpu.make_async_copy(
                table_hbm.at[idx_v.at[pl.ds(c * _CHUNK, _CHUNK)]],
                bufs[p], gsems[p])
            h.start()
            return h

        def store(c):
            p = c % 2
            h = pltpu.make_async_copy(
                bufs[p], out_hbm.at[pl.ds(base + c * _CHUNK, _CHUNK)],
                ssems[p])
            h.start()
            return h

        g_h = [None] * _N_CHUNKS
        s_h = [None] * _N_CHUNKS
        g_h[0] = gather(0)
        g_h[1] = gather(1)
        for c in range(_N_CHUNKS):
            g_h[c].wait()
            s_h[c] = store(c)
            if c + 2 < _N_CHUNKS:
                s_h[c].wait()
                g_h[c + 2] = gather(c + 2)
        s_h[_N_CHUNKS - 2].wait()
        s_h[_N_CHUNKS - 1].wait()

    return emb_kernel


_emb_lookup = _make_emb_kernel()


def kernel(idx, embedding_table):
    idx1 = idx.reshape(-1).astype(jnp.int32)
    table_p = jnp.pad(embedding_table, ((0, 0), (0, DPAD - VOCAB)))
    return _emb_lookup(idx1, table_p)[:, :VOCAB]
